# baseline (speedup 1.0000x reference)
import functools
import jax
import jax.numpy as jnp
from jax import lax
from jax.experimental import pallas as pl
from jax.experimental.pallas import tpu as pltpu
from jax.experimental.pallas import tpu_sc as plsc

D = 256
NB = 2000
NC, NS, L = 2, 16, 16

E_EDGES = 160000
NPAIR = 2 * E_EDGES
NUM_REL2 = 10000
ROWS = 10240
TRASH = 10000
BP = 80
EPAD = 161280
TKEYS = 100_000_000
SH = 4
BLK = 512


def _rowspec(cols):
    return pl.BlockSpec((NB, cols), lambda i: (i, 0))


def _accspec(cols):
    return pl.BlockSpec((SH, NB, cols), lambda i: (0, i, 0))


def _fullspec(shape):
    return pl.BlockSpec(shape, lambda i: tuple(0 for _ in shape))


def _normalize_rows(x):
    n = jnp.sqrt(jnp.sum(x * x, axis=1, keepdims=True))
    return x / jnp.maximum(n, 1e-12)


def _shsum(ref):
    x = ref[0]
    for k in range(1, SH):
        x = x + ref[k]
    return x


def _norm_body(x_ref, o_ref):
    o_ref[...] = _normalize_rows(x_ref[...])


def _tc_norm(x):
    return pl.pallas_call(
        _norm_body,
        grid=(x.shape[0] // NB,),
        in_specs=[_rowspec(D)],
        out_specs=_rowspec(D),
        out_shape=jax.ShapeDtypeStruct(x.shape, x.dtype),
    )(x)


def _relstep_body(rs_ref, ct_ref, r0_ref, relp_ref,
                  wir_ref, wic_ref, whh_ref, bih_ref, bhh_ref, o_ref):
    relsum = _shsum(rs_ref)
    cnt = _shsum(ct_ref)[:, 0:1]
    cur = relsum / jnp.maximum(cnt, 1.0)
    gi = (jnp.dot(r0_ref[...], wir_ref[...], preferred_element_type=jnp.float32)
          + jnp.dot(cur, wic_ref[...], preferred_element_type=jnp.float32)
          + bih_ref[...])
    gh = (jnp.dot(relp_ref[...], whh_ref[...], preferred_element_type=jnp.float32)
          + bhh_ref[...])
    i_r, i_z, i_n = gi[:, :D], gi[:, D:2 * D], gi[:, 2 * D:]
    h_r, h_z, h_n = gh[:, :D], gh[:, D:2 * D], gh[:, 2 * D:]
    r = jax.nn.sigmoid(i_r + h_r)
    z = jax.nn.sigmoid(i_z + h_z)
    n = jnp.tanh(i_n + r * h_n)
    o_ref[...] = _normalize_rows((1.0 - z) * n + z * relp_ref[...])


def _tc_relstep(rs, ct, r0, relp, wir, wic, whh, bih, bhh):
    m = r0.shape[0]
    return pl.pallas_call(
        _relstep_body,
        grid=(m // NB,),
        in_specs=[_accspec(D), _accspec(8),
                  _rowspec(D), _rowspec(D),
                  _fullspec((D, 3 * D)), _fullspec((D, 3 * D)), _fullspec((D, 3 * D)),
                  _fullspec((1, 3 * D)), _fullspec((1, 3 * D))],
        out_specs=_rowspec(D),
        out_shape=jax.ShapeDtypeStruct((m, D), jnp.float32),
    )(rs, ct, r0, relp, wir, wic, whh, bih, bhh)


def _prep_body(h_ref, rel_ref, wn_ref, wl_ref, hw_ref, relw_ref, hlw_ref):
    hw_ref[...] = jnp.dot(h_ref[...], wn_ref[...], preferred_element_type=jnp.float32)
    relw_ref[...] = jnp.dot(rel_ref[...], wn_ref[...], preferred_element_type=jnp.float32)
    hlw_ref[...] = jnp.dot(h_ref[...], wl_ref[...], preferred_element_type=jnp.float32)


def _tc_prep(h, rel, wn, wl):
    m = h.shape[0]
    sd = jax.ShapeDtypeStruct((m, D), jnp.float32)
    return pl.pallas_call(
        _prep_body,
        grid=(m // NB,),
        in_specs=[_rowspec(D), _rowspec(D), _fullspec((D, D)), _fullspec((D, D))],
        out_specs=(_rowspec(D), _rowspec(D), _rowspec(D)),
        out_shape=(sd, sd, sd),
    )(h, rel, wn, wl)


def _prep2_body(a_ref, d_ref, hlw_ref, rel_ref, wn_ref, wl_ref,
                hw_ref, relw_ref, hlw2_ref):
    acc = _shsum(a_ref)
    deg = _shsum(d_ref)[:, 0:1]
    h2 = jax.nn.relu(acc / jnp.maximum(deg, 1.0) + hlw_ref[...])
    hw_ref[...] = jnp.dot(h2, wn_ref[...], preferred_element_type=jnp.float32)
    relw_ref[...] = jnp.dot(rel_ref[...], wn_ref[...], preferred_element_type=jnp.float32)
    hlw2_ref[...] = jnp.dot(h2, wl_ref[...], preferred_element_type=jnp.float32)


def _tc_prep2(a, d, hlw, rel, wn, wl):
    m = hlw.shape[0]
    sd = jax.ShapeDtypeStruct((m, D), jnp.float32)
    return pl.pallas_call(
        _prep2_body,
        grid=(m // NB,),
        in_specs=[_accspec(D), _accspec(8), _rowspec(D),
                  _rowspec(D), _fullspec((D, D)), _fullspec((D, D))],
        out_specs=(_rowspec(D), _rowspec(D), _rowspec(D)),
        out_shape=(sd, sd, sd),
    )(a, d, hlw, rel, wn, wl)


def _final_body(a_ref, d_ref, hlw_ref, ent_ref, gw_ref, gb_ref, o_ref):
    acc = _shsum(a_ref)
    deg = _shsum(d_ref)[:, 0:1]
    cur = _normalize_rows(
        jax.nn.relu(acc / jnp.maximum(deg, 1.0) + hlw_ref[...]))
    gate = jax.nn.sigmoid(
        jnp.dot(ent_ref[...], gw_ref[...], preferred_element_type=jnp.float32)
        + gb_ref[...])
    o_ref[...] = gate * cur + (1.0 - gate) * ent_ref[...]


def _tc_final(a, d, hlw, ent, gw, gb):
    m = hlw.shape[0]
    return pl.pallas_call(
        _final_body,
        grid=(m // NB,),
        in_specs=[_accspec(D), _accspec(8), _rowspec(D),
                  _rowspec(D), _fullspec((D, D)), _fullspec((1, D))],
        out_specs=_rowspec(D),
        out_shape=jax.ShapeDtypeStruct((m, D), jnp.float32),
    )(a, d, hlw, ent, gw, gb)


def _acc_body_factory(use_win):
    def body(*refs):
        if use_win:
            idx_ref, win_ref, msg_ref, acc_ref = refs
        else:
            idx_ref, msg_ref, acc_ref = refs
            win_ref = None

        @pl.when(pl.program_id(0) == 0)
        def _():
            acc_ref[...] = jnp.zeros((SH, ROWS, D), jnp.float32)

        def it(m, c):
            for k in range(SH):
                i = m * SH + k
                d = idx_ref[i]
                row = msg_ref[pl.ds(i, 1), :]
                if use_win:
                    w = win_ref[i]
                    row = row * w
                acc_ref[k, pl.ds(d, 1), :] = acc_ref[k, pl.ds(d, 1), :] + row
            return c

        lax.fori_loop(0, BLK // SH, it, 0)
    return body


def _tc_segsum(idx, msg, win):
    n = msg.shape[0]
    use_win = win is not None
    in_specs = [pl.BlockSpec((BLK,), lambda i: (i,), memory_space=pltpu.SMEM)]
    args = [idx]
    if use_win:
        in_specs.append(pl.BlockSpec((BLK,), lambda i: (i,),
                                     memory_space=pltpu.SMEM))
        args.append(win)
    in_specs.append(pl.BlockSpec((BLK, D), lambda i: (i, 0)))
    args.append(msg)
    acc_sd = jax.ShapeDtypeStruct((SH, ROWS, D), jnp.float32)
    acc_spec = pl.BlockSpec((SH, ROWS, D), lambda i: (0, 0, 0))
    return pl.pallas_call(
        _acc_body_factory(use_win),
        grid=(n // BLK,),
        in_specs=in_specs,
        out_specs=acc_spec,
        out_shape=acc_sd,
    )(*args)


def _cnt_body_factory(use_win):
    def body(*refs):
        if use_win:
            idx_ref, win_ref, cnt_ref = refs
        else:
            idx_ref, cnt_ref = refs
            win_ref = None

        @pl.when(pl.program_id(0) == 0)
        def _():
            cnt_ref[...] = jnp.zeros((SH, ROWS, 8), jnp.float32)

        def it(m, c):
            for k in range(SH):
                i = m * SH + k
                d = idx_ref[i]
                wv = win_ref[i] if use_win else 1.0
                cnt_ref[k, pl.ds(d, 1), :] = cnt_ref[k, pl.ds(d, 1), :] + wv
            return c

        lax.fori_loop(0, BLK // SH, it, 0)
    return body


def _tc_segcnt(idx, win, n):
    use_win = win is not None
    in_specs = [pl.BlockSpec((BLK,), lambda i: (i,), memory_space=pltpu.SMEM)]
    args = [idx]
    if use_win:
        in_specs.append(pl.BlockSpec((BLK,), lambda i: (i,),
                                     memory_space=pltpu.SMEM))
        args.append(win)
    cnt_sd = jax.ShapeDtypeStruct((SH, ROWS, 8), jnp.float32)
    cnt_spec = pl.BlockSpec((SH, ROWS, 8), lambda i: (0, 0, 0))
    return pl.pallas_call(
        _cnt_body_factory(use_win),
        grid=(n // BLK,),
        in_specs=in_specs,
        out_specs=cnt_spec,
        out_shape=cnt_sd,
    )(*args)


def _sc_mesh():
    return plsc.VectorSubcoreMesh(core_axis_name="c", subcore_axis_name="s",
                                  num_cores=NC, num_subcores=NS)


def _wid():
    return lax.axis_index("c") * NS + lax.axis_index("s")


def _iota16():
    return lax.iota(jnp.int32, L)


def _sc_a1_body(e_all, r_all, t_out, keys_out,
                ebuf0, rbuf0, ebuf1, rbuf1, keybuf, valbuf, sem0, sem1):
    wid = _wid()
    base = wid * (NPAIR // (NC * NS))
    nblk = NPAIR // (NC * NS) // BP
    sets = ((ebuf0, rbuf0, sem0), (ebuf1, rbuf1, sem1))

    def issue(b, j):
        eb, rb, sm = sets[j]
        off = base + b * BP
        pltpu.async_copy(e_all.at[pl.ds(off, BP)], eb, sm)
        pltpu.async_copy(r_all.at[pl.ds(off, BP)], rb, sm)

    def process(b, j):
        eb, rb, sm = sets[j]
        pltpu.make_async_copy(e_all.at[pl.ds(0, BP)], eb, sm).wait()
        pltpu.make_async_copy(r_all.at[pl.ds(0, BP)], rb, sm).wait()
        off = base + b * BP
        for g in range(BP // L):
            sl = pl.ds(g * L, L)
            keybuf[sl] = eb[sl] * NUM_REL2 + rb[sl]
            valbuf[sl] = off + g * L + _iota16()
        pltpu.sync_copy(keybuf, keys_out.at[pl.ds(off, BP)])
        pltpu.sync_copy(valbuf, t_out.at[keybuf])

    issue(0, 0)

    def pairbody(p, c):
        b0 = 2 * p
        issue(b0 + 1, 1)
        process(b0, 0)

        @pl.when(b0 + 2 < nblk)
        def _():
            issue(b0 + 2, 0)
        process(b0 + 1, 1)
        return c

    lax.fori_loop(0, nblk // 2, pairbody, 0)
    if nblk % 2:
        process(nblk - 1, 0)


def _sc_a1(e_all, r_all):
    idx = pltpu.VMEM((BP,), jnp.int32)
    f = pl.kernel(
        _sc_a1_body,
        out_type=(jax.ShapeDtypeStruct((TKEYS,), jnp.int32),
                  jax.ShapeDtypeStruct((NPAIR,), jnp.int32)),
        mesh=_sc_mesh(),
        scratch_types=[idx, idx, idx, idx, idx, idx,
                       pltpu.SemaphoreType.DMA, pltpu.SemaphoreType.DMA],
    )
    return f(e_all, r_all)


def _sc_a2_body(keys, e_all, t_in, ent, msg_out, win_out,
                keybuf0, ebuf0, tbuf0, entrows0,
                keybuf1, ebuf1, tbuf1, entrows1,
                winbuf, sem0, sem1):
    wid = _wid()
    base = wid * (NPAIR // (NC * NS))
    nblk = NPAIR // (NC * NS) // BP
    sets = ((keybuf0, ebuf0, tbuf0, entrows0, sem0),
            (keybuf1, ebuf1, tbuf1, entrows1, sem1))

    def issue(b, j):
        kb, eb, tb, er, sm = sets[j]
        off = base + b * BP
        pltpu.sync_copy(keys.at[pl.ds(off, BP)], kb)
        pltpu.sync_copy(e_all.at[pl.ds(off, BP)], eb)
        pltpu.async_copy(t_in.at[kb], tb, sm)
        pltpu.async_copy(ent.at[eb], er, sm)

    def process(b, j):
        kb, eb, tb, er, sm = sets[j]
        pltpu.make_async_copy(t_in.at[kb], tb, sm).wait()
        pltpu.make_async_copy(ent.at[eb], er, sm).wait()
        off = base + b * BP
        for g in range(BP // L):
            sl = pl.ds(g * L, L)
            pid = off + g * L + _iota16()
            winbuf[sl] = jnp.where(tb[sl] == pid, 1.0, 0.0)
        pltpu.sync_copy(er, msg_out.at[pl.ds(off, BP)])
        pltpu.sync_copy(winbuf, win_out.at[pl.ds(off, BP)])

    issue(0, 0)

    def pairbody(p, c):
        b0 = 2 * p
        issue(b0 + 1, 1)
        process(b0, 0)

        @pl.when(b0 + 2 < nblk)
        def _():
            issue(b0 + 2, 0)
        process(b0 + 1, 1)
        return c

    lax.fori_loop(0, nblk // 2, pairbody, 0)
    if nblk % 2:
        process(nblk - 1, 0)


def _sc_a2(keys, e_all, t_in, ent):
    idx = pltpu.VMEM((BP,), jnp.int32)
    rows = pltpu.VMEM((BP, D), jnp.float32)
    f = pl.kernel(
        _sc_a2_body,
        out_type=(jax.ShapeDtypeStruct((NPAIR, D), jnp.float32),
                  jax.ShapeDtypeStruct((NPAIR,), jnp.float32)),
        mesh=_sc_mesh(),
        scratch_types=[idx, idx, idx, rows,
                       idx, idx, idx, rows,
                       pltpu.VMEM((BP,), jnp.float32),
                       pltpu.SemaphoreType.DMA, pltpu.SemaphoreType.DMA],
    )
    return f(keys, e_all, t_in, ent)


def _sc_edge_msg_body(srcp, rtp, hw, relw, msg_out,
                      sbuf0, tbuf0, rows1_0, rows2_0,
                      sbuf1, tbuf1, rows1_1, rows2_1, sem0, sem1):
    wid = _wid()
    base = wid * (EPAD // (NC * NS))
    nblk = EPAD // (NC * NS) // BP
    sets = ((sbuf0, tbuf0, rows1_0, rows2_0, sem0),
            (sbuf1, tbuf1, rows1_1, rows2_1, sem1))

    def issue(b, j):
        sb, tb, r1, r2, sm = sets[j]
        off = base + b * BP
        pltpu.sync_copy(srcp.at[pl.ds(off, BP)], sb)
        pltpu.sync_copy(rtp.at[pl.ds(off, BP)], tb)
        pltpu.async_copy(hw.at[sb], r1, sm)
        pltpu.async_copy(relw.at[tb], r2, sm)

    def process(b, j):
        sb, tb, r1, r2, sm = sets[j]
        pltpu.make_async_copy(hw.at[sb], r1, sm).wait()
        pltpu.make_async_copy(relw.at[tb], r2, sm).wait()
        off = base + b * BP

        def radd(i, c2):
            for g in range(D // L):
                sl = pl.ds(g * L, L)
                r1[i, sl] = r1[i, sl] + r2[i, sl]
            return c2

        lax.fori_loop(0, BP, radd, 0)
        pltpu.sync_copy(r1, msg_out.at[pl.ds(off, BP)])

    issue(0, 0)

    def pairbody(p, c):
        b0 = 2 * p
        issue(b0 + 1, 1)
        process(b0, 0)

        @pl.when(b0 + 2 < nblk)
        def _():
            issue(b0 + 2, 0)
        process(b0 + 1, 1)
        return c

    lax.fori_loop(0, nblk // 2, pairbody, 0)
    if nblk % 2:
        process(nblk - 1, 0)


def _sc_edge_msg(srcp, rtp, hw, relw):
    idx = pltpu.VMEM((BP,), jnp.int32)
    rows = pltpu.VMEM((BP, D), jnp.float32)
    f = pl.kernel(
        _sc_edge_msg_body,
        out_type=jax.ShapeDtypeStruct((EPAD, D), jnp.float32),
        mesh=_sc_mesh(),
        scratch_types=[idx, idx, rows, rows,
                       idx, idx, rows, rows,
                       pltpu.SemaphoreType.DMA, pltpu.SemaphoreType.DMA],
    )
    return f(srcp, rtp, hw, relw)


def kernel(edges, static_entity_embed, static_relation_embed, gate_weight, gate_bias,
           gru_w_ih, gru_w_hh, gru_b_ih, gru_b_hh, rgcn_w_neigh, rgcn_w_loop):
    num_layer = rgcn_w_neigh.shape[0]
    R0 = static_relation_embed
    wir = gru_w_ih[:, :D].T
    wic = gru_w_ih[:, D:].T
    whh = gru_w_hh.T
    bih = gru_b_ih.reshape(1, 3 * D)
    bhh = gru_b_hh.reshape(1, 3 * D)
    gb = gate_bias.reshape(1, D)
    padn = EPAD - E_EDGES
    pad0 = jnp.zeros((padn,), jnp.int32)
    padt = jnp.full((padn,), TRASH, jnp.int32)

    ent = _tc_norm(static_entity_embed)
    rel = R0
    for t in range(edges.shape[0]):
        edge = edges[t]
        src, rtype, dst = edge[:, 0], edge[:, 1], edge[:, 2]
        e_all = jnp.concatenate([src, dst])
        r_all = jnp.concatenate([rtype, rtype])
        t_tab, keys = _sc_a1(e_all, r_all)
        amsg, win = _sc_a2(keys, e_all, t_tab, ent)
        rs = _tc_segsum(r_all, amsg, win)
        ct = _tc_segcnt(r_all, win, NPAIR)
        rel = _tc_relstep(rs, ct, R0, rel, wir, wic, whh, bih, bhh)
        srcp = jnp.concatenate([src, pad0])
        rtp = jnp.concatenate([rtype, pad0])
        dstp = jnp.concatenate([dst, padt])
        hw, relw, hlw = _tc_prep(ent, rel, rgcn_w_neigh[0], rgcn_w_loop[0])
        emsg = _sc_edge_msg(srcp, rtp, hw, relw)
        a = _tc_segsum(dstp, emsg, None)
        d = _tc_segcnt(dstp, None, EPAD)
        for l in range(1, num_layer):
            hw, relw, hlw = _tc_prep2(a, d, hlw, rel,
                                      rgcn_w_neigh[l], rgcn_w_loop[l])
            emsg = _sc_edge_msg(srcp, rtp, hw, relw)
            a = _tc_segsum(dstp, emsg, None)
        ent = _tc_final(a, d, hlw, ent, gate_weight, gb)
    ent = _tc_norm(ent)
    return ent, rel
